# Initial kernel scaffold; baseline (speedup 1.0000x reference)
import jax
import jax.numpy as jnp
from jax.experimental import pallas as pl


def kernel(input_feat, pos_emb, ln_weight, ln_bias):
    raise NotImplementedError("write your pallas kernel here")


# speedup vs baseline: 3.5212x; 3.5212x over previous
import functools

import jax
import jax.numpy as jnp
from jax.experimental import pallas as pl
from jax.experimental.pallas import tpu as pltpu

ROWS = 512


def _ln_block(input_ref, pos_ref, w_ref, b_ref, out_ref):
    x = input_ref[0] + pos_ref[...]
    mean = jnp.mean(x, axis=-1, keepdims=True)
    cent = x - mean
    var = jnp.mean(cent * cent, axis=-1, keepdims=True)
    normed = cent * jax.lax.rsqrt(var + 1e-5)
    out_ref[0] = normed * w_ref[...] + b_ref[...]


@functools.partial(jax.jit, static_argnames=())
def kernel(input_feat, pos_emb, ln_weight, ln_bias):
    bsz, seq, hid = input_feat.shape
    rows = ROWS if seq % ROWS == 0 else seq
    grid = (seq // rows, bsz)
    return pl.pallas_call(
        _ln_block,
        grid=grid,
        in_specs=[
            pl.BlockSpec((1, rows, hid), lambda s, b: (b, s, 0)),
            pl.BlockSpec((rows, hid), lambda s, b: (s, 0)),
            pl.BlockSpec((hid,), lambda s, b: (0,)),
            pl.BlockSpec((hid,), lambda s, b: (0,)),
        ],
        out_specs=pl.BlockSpec((1, rows, hid), lambda s, b: (b, s, 0)),
        out_shape=jax.ShapeDtypeStruct((bsz, seq, hid), input_feat.dtype),
        compiler_params=pltpu.CompilerParams(
            dimension_semantics=("arbitrary", "arbitrary"),
        ),
    )(input_feat, pos_emb[:seq], ln_weight, ln_bias)


# speedup vs baseline: 4.0159x; 1.1405x over previous
import functools

import jax
import jax.numpy as jnp
from jax.experimental import pallas as pl
from jax.experimental.pallas import tpu as pltpu

ROWS = 1024


def _ln_block(input_ref, pos_ref, w_ref, b_ref, out_ref):
    x = input_ref[0] + pos_ref[...]
    mean = jnp.mean(x, axis=-1, keepdims=True)
    cent = x - mean
    var = jnp.mean(cent * cent, axis=-1, keepdims=True)
    normed = cent * jax.lax.rsqrt(var + 1e-5)
    out_ref[0] = normed * w_ref[...] + b_ref[...]


@functools.partial(jax.jit, static_argnames=())
def kernel(input_feat, pos_emb, ln_weight, ln_bias):
    bsz, seq, hid = input_feat.shape
    rows = ROWS if seq % ROWS == 0 else seq
    grid = (seq // rows, bsz)
    return pl.pallas_call(
        _ln_block,
        grid=grid,
        in_specs=[
            pl.BlockSpec((1, rows, hid), lambda s, b: (b, s, 0)),
            pl.BlockSpec((rows, hid), lambda s, b: (s, 0)),
            pl.BlockSpec((hid,), lambda s, b: (0,)),
            pl.BlockSpec((hid,), lambda s, b: (0,)),
        ],
        out_specs=pl.BlockSpec((1, rows, hid), lambda s, b: (b, s, 0)),
        out_shape=jax.ShapeDtypeStruct((bsz, seq, hid), input_feat.dtype),
        compiler_params=pltpu.CompilerParams(
            dimension_semantics=("arbitrary", "arbitrary"),
        ),
    )(input_feat, pos_emb[:seq], ln_weight, ln_bias)


# speedup vs baseline: 4.1703x; 1.0385x over previous
import functools

import jax
import jax.numpy as jnp
from jax.experimental import pallas as pl
from jax.experimental.pallas import tpu as pltpu

ROWS = 2048


def _ln_block(input_ref, pos_ref, w_ref, b_ref, out_ref):
    x = input_ref[0] + pos_ref[...]
    mean = jnp.mean(x, axis=-1, keepdims=True)
    cent = x - mean
    var = jnp.mean(cent * cent, axis=-1, keepdims=True)
    normed = cent * jax.lax.rsqrt(var + 1e-5)
    out_ref[0] = normed * w_ref[...] + b_ref[...]


@functools.partial(jax.jit, static_argnames=())
def kernel(input_feat, pos_emb, ln_weight, ln_bias):
    bsz, seq, hid = input_feat.shape
    rows = ROWS if seq % ROWS == 0 else seq
    grid = (seq // rows, bsz)
    return pl.pallas_call(
        _ln_block,
        grid=grid,
        in_specs=[
            pl.BlockSpec((1, rows, hid), lambda s, b: (b, s, 0)),
            pl.BlockSpec((rows, hid), lambda s, b: (s, 0)),
            pl.BlockSpec((hid,), lambda s, b: (0,)),
            pl.BlockSpec((hid,), lambda s, b: (0,)),
        ],
        out_specs=pl.BlockSpec((1, rows, hid), lambda s, b: (b, s, 0)),
        out_shape=jax.ShapeDtypeStruct((bsz, seq, hid), input_feat.dtype),
        compiler_params=pltpu.CompilerParams(
            dimension_semantics=("arbitrary", "arbitrary"),
        ),
    )(input_feat, pos_emb[:seq], ln_weight, ln_bias)
